# Initial kernel scaffold; baseline (speedup 1.0000x reference)
import jax
import jax.numpy as jnp
from jax.experimental import pallas as pl


def kernel(atom_x, atom_edge_index, atom_edge_attr, atom_batch, fg_x, fg_edge_index, fg_edge_attr, fg_x_batch, atom2fg_index, params):
    raise NotImplementedError("write your pallas kernel here")


# speedup vs baseline: 2.4450x; 2.4450x over previous
import functools

import jax
import jax.numpy as jnp
from jax import lax
from jax.experimental import pallas as pl
from jax.experimental.pallas import tpu as pltpu
import jax.experimental.pallas.tpu_sc as plsc

_NC, _NS = 2, 16
_NW = _NC * _NS
_D = 128
_F32 = jnp.float32


def _linear(x, w, b):
    n, _ = x.shape
    dout = w.shape[1]

    def body(x_ref, w_ref, b_ref, o_ref):
        o_ref[...] = (
            jnp.dot(x_ref[...], w_ref[...], preferred_element_type=_F32,
                    precision=lax.Precision.HIGHEST)
            + b_ref[...]
        )

    return pl.pallas_call(
        body,
        out_shape=jax.ShapeDtypeStruct((n, dout), _F32),
    )(x, w, b.reshape(1, dout))


def _edge_embed(attr, w3, b3, bm):
    nl, din, _ = w3.shape
    e = attr.shape[0]
    nb = e // bm

    def body(a_ref, w_ref, b_ref, o_ref):
        o_ref[...] = (
            jnp.dot(a_ref[...], w_ref[0], preferred_element_type=_F32,
                    precision=lax.Precision.HIGHEST)
            + b_ref[0]
        )[None]

    return pl.pallas_call(
        body,
        grid=(nl, nb),
        in_specs=[
            pl.BlockSpec((bm, din), lambda i, j: (j, 0)),
            pl.BlockSpec((1, din, _D), lambda i, j: (i, 0, 0)),
            pl.BlockSpec((1, 1, _D), lambda i, j: (i, 0, 0)),
        ],
        out_specs=pl.BlockSpec((1, bm, _D), lambda i, j: (i, j, 0)),
        out_shape=jax.ShapeDtypeStruct((nl, e, _D), _F32),
    )(attr, w3, b3)


def _gine_mlp(x, parts, gp, bn, relu_out):
    n = x.shape[0]

    def body(x_ref, p_ref, w1_ref, b1_ref, g1_ref, bb1_ref,
             w2_ref, b2_ref, g2_ref, bb2_ref, o_ref):
        h0 = x_ref[...] + (p_ref[0, :n, :] + p_ref[1, :n, :])
        h = jnp.dot(h0, w1_ref[...], preferred_element_type=_F32) + b1_ref[...]
        mu = jnp.mean(h, axis=0, keepdims=True)
        var = jnp.mean((h - mu) ** 2, axis=0, keepdims=True)
        h = g1_ref[...] * (h - mu) / jnp.sqrt(var + 1e-5) + bb1_ref[...]
        h = jnp.maximum(h, 0.0)
        y = jnp.dot(h, w2_ref[...], preferred_element_type=_F32) + b2_ref[...]
        mu2 = jnp.mean(y, axis=0, keepdims=True)
        var2 = jnp.mean((y - mu2) ** 2, axis=0, keepdims=True)
        y = g2_ref[...] * (y - mu2) / jnp.sqrt(var2 + 1e-5) + bb2_ref[...]
        if relu_out:
            y = jnp.maximum(y, 0.0)
        o_ref[...] = y

    d2 = gp["lin1"]["W"].shape[1]
    return pl.pallas_call(
        body,
        out_shape=jax.ShapeDtypeStruct((n, _D), _F32),
    )(x, parts,
      gp["lin1"]["W"], gp["lin1"]["b"].reshape(1, d2),
      gp["g1"].reshape(1, d2), gp["b1"].reshape(1, d2),
      gp["lin2"]["W"], gp["lin2"]["b"].reshape(1, _D),
      bn["g"].reshape(1, _D), bn["b"].reshape(1, _D))


def _a2f_combine(fx, parts, f_idx2d, wa, ba, n_fg):
    rows, cols = f_idx2d.shape

    def body(fx_ref, p_ref, fi_ref, wa_ref, ba_ref, o_ref):
        sums = p_ref[0, :n_fg, :] + p_ref[1, :n_fg, :]
        iot = lax.broadcasted_iota(jnp.int32, (n_fg, 1), 0)
        cnt = jnp.zeros((n_fg, 1), _F32)
        for j in range(rows):
            blk = fi_ref[j]
            cnt = cnt + jnp.sum((iot == blk[None, :]).astype(_F32),
                                axis=1, keepdims=True)
        mean = sums / jnp.maximum(cnt, 1.0)
        o_ref[...] = (
            fx_ref[...]
            + jnp.dot(mean, wa_ref[...], preferred_element_type=_F32)
            + ba_ref[...]
        )

    return pl.pallas_call(
        body,
        out_shape=jax.ShapeDtypeStruct((n_fg, _D), _F32),
    )(fx, parts, f_idx2d, wa, ba.reshape(1, _D))


def _pool(fx, batch2d, g):
    rows, cols = batch2d.shape

    def body(fx_ref, b_ref, o_ref):
        iot = lax.broadcasted_iota(jnp.int32, (g, 1), 0)
        psum = jnp.zeros((g, _D), _F32)
        pcnt = jnp.zeros((g, 1), _F32)
        for j in range(rows):
            oh = (iot == b_ref[j][None, :]).astype(_F32)
            psum = psum + jnp.dot(oh, fx_ref[pl.ds(j * cols, cols), :],
                                  preferred_element_type=_F32,
                                  precision=lax.Precision.HIGHEST)
            pcnt = pcnt + jnp.sum(oh, axis=1, keepdims=True)
        o_ref[...] = psum / jnp.maximum(pcnt, 1.0)

    n = fx.shape[0]
    return pl.pallas_call(
        body,
        out_shape=jax.ShapeDtypeStruct((g, _D), _F32),
    )(fx, batch2d)


def _sc_segsum(table, src3, dst3, n_acc, ea=None, layer=0):
    nw, nch, k = src3.shape
    epw = nch * k
    rp = n_acc // _NS
    zr = min(k, rp)
    with_ea = ea is not None
    mesh = plsc.VectorSubcoreMesh(core_axis_name="c", subcore_axis_name="s",
                                  num_cores=_NC, num_subcores=_NS)

    def body(*refs):
        if with_ea:
            (src_hbm, dst_hbm, ea_hbm, table_hbm, out_hbm,
             acc_sh, srcall, dstall, rows_v, ea_v, sem) = refs
        else:
            (src_hbm, dst_hbm, table_hbm, out_hbm,
             acc_sh, srcall, dstall, rows_v, sem) = refs
        cid = lax.axis_index("c")
        sid = lax.axis_index("s")
        wid = sid * _NC + cid

        def zrow(r, c_):
            for c in range(_D // 16):
                rows_v[r, pl.ds(c * 16, 16)] = jnp.zeros((16,), _F32)
            return c_
        lax.fori_loop(0, zr, zrow, 0)
        off = 0
        while off < rp:
            take = min(zr, rp - off)
            pltpu.sync_copy(rows_v.at[pl.ds(0, take)],
                            acc_sh.at[pl.ds(sid * rp + off, take)])
            off += take
        plsc.subcore_barrier()

        pltpu.sync_copy(src_hbm.at[wid], srcall)
        pltpu.sync_copy(dst_hbm.at[wid], dstall)

        def chunk(i, c_):
            pltpu.async_copy(table_hbm.at[srcall.at[pl.ds(i * k, k)]],
                             rows_v, sem).wait()
            if with_ea:
                base = wid * epw + i * k
                pltpu.sync_copy(ea_hbm.at[layer, pl.ds(base, k)], ea_v)

                def rowfn(r, c2_):
                    for c in range(_D // 16):
                        s = pl.ds(c * 16, 16)
                        rows_v[r, s] = jnp.maximum(rows_v[r, s] + ea_v[r, s],
                                                   0.0)
                    return c2_
                lax.fori_loop(0, k, rowfn, 0)
            pltpu.sync_copy(rows_v, acc_sh.at[dstall.at[i]], add=True)
            return c_
        lax.fori_loop(0, nch, chunk, 0)

        plsc.subcore_barrier()
        off = 0
        while off < rp:
            take = min(zr, rp - off)
            pltpu.sync_copy(acc_sh.at[pl.ds(sid * rp + off, take)],
                            out_hbm.at[cid, pl.ds(sid * rp + off, take)])
            off += take

    scratch = [
        pltpu.VMEM_SHARED((n_acc, _D), _F32),
        pltpu.VMEM((epw,), jnp.int32),
        pltpu.VMEM((nch, k), jnp.int32),
        pltpu.VMEM((k, _D), _F32),
    ]
    if with_ea:
        scratch.append(pltpu.VMEM((k, _D), _F32))
    scratch.append(pltpu.SemaphoreType.DMA)

    f = pl.kernel(body,
                  out_type=jax.ShapeDtypeStruct((_NC, n_acc, _D), _F32),
                  mesh=mesh, scratch_types=scratch)
    src2 = src3.reshape(nw, epw)
    if with_ea:
        return f(src2, dst3, ea, table)
    return f(src2, dst3, table)


def kernel(atom_x, atom_edge_index, atom_edge_attr, atom_batch, fg_x,
           fg_edge_index, fg_edge_attr, fg_x_batch, atom2fg_index, params):
    p = params
    n_atom = atom_x.shape[0]
    e_atom = atom_edge_index.shape[1]
    n_fg = fg_x.shape[0]
    e_fg = fg_edge_index.shape[1]
    a2f = atom2fg_index.shape[1]
    g = 64

    ax = _linear(atom_x, p["atom_emb"]["W"], p["atom_emb"]["b"])
    fx = _linear(fg_x, p["fg_emb"]["W"], p["fg_emb"]["b"])

    w3 = jnp.stack([l["W"] for l in p["bond_emb"]])
    b3 = jnp.stack([l["b"] for l in p["bond_emb"]])[:, None, :]
    ea3 = _edge_embed(atom_edge_attr, w3, b3, bm=6400)

    wf = jnp.stack([l["W"] for l in p["fg_edge_emb"]])
    bf = jnp.stack([l["b"] for l in p["fg_edge_emb"]])[:, None, :]
    fea2 = _edge_embed(fg_edge_attr, wf, bf, bm=8000)

    k_at = 80
    nch_at = e_atom // _NW // k_at
    src_at = atom_edge_index[0].reshape(_NW, nch_at, k_at)
    dst_at = atom_edge_index[1].reshape(_NW, nch_at, k_at)
    n_acc_at = -(-n_atom // 128) * 128
    for i in range(3):
        parts = _sc_segsum(ax, src_at, dst_at, n_acc_at, ea=ea3, layer=i)
        ax = _gine_mlp(ax, parts, p["atom_gin"][i], p["atom_bn"][i],
                       relu_out=(i != 2))

    k_a2f = 80
    pad = (-a2f) % (_NW * k_a2f)
    a_idx = jnp.concatenate(
        [atom2fg_index[0], jnp.zeros((pad,), jnp.int32)])
    f_idx = jnp.concatenate(
        [atom2fg_index[1], jnp.full((pad,), n_fg, jnp.int32)])
    nch_a2f = (a2f + pad) // _NW // k_a2f
    n_acc = 2048
    parts = _sc_segsum(ax, a_idx.reshape(_NW, nch_a2f, k_a2f),
                       f_idx.reshape(_NW, nch_a2f, k_a2f), n_acc)
    fx = _a2f_combine(fx, parts, atom2fg_index[1].reshape(8, a2f // 8),
                      p["a2f_lin"]["W"], p["a2f_lin"]["b"], n_fg)

    k_fg = 40
    nch_fg = e_fg // _NW // k_fg
    src_fg = fg_edge_index[0].reshape(_NW, nch_fg, k_fg)
    dst_fg = fg_edge_index[1].reshape(_NW, nch_fg, k_fg)
    n_acc_fg = -(-n_fg // 128) * 128
    for i in range(2):
        parts = _sc_segsum(fx, src_fg, dst_fg, n_acc_fg, ea=fea2, layer=i)
        fx = _gine_mlp(fx, parts, p["fg_gin"][i], p["fg_bn"][i],
                       relu_out=(i != 1))

    return _pool(fx, fg_x_batch.reshape(8, n_fg // 8), g)


# speedup vs baseline: 2.4487x; 1.0015x over previous
import jax
import jax.numpy as jnp
from jax import lax
from jax.experimental import pallas as pl
from jax.experimental.pallas import tpu as pltpu
import jax.experimental.pallas.tpu_sc as plsc

_NC, _NS = 2, 16
_NW = _NC * _NS
_D = 128
_F32 = jnp.float32


def _linear(x, w, b):
    n, _ = x.shape
    dout = w.shape[1]

    def body(x_ref, w_ref, b_ref, o_ref):
        o_ref[...] = (
            jnp.dot(x_ref[...], w_ref[...], preferred_element_type=_F32,
                    precision=lax.Precision.HIGHEST)
            + b_ref[...]
        )

    return pl.pallas_call(
        body,
        out_shape=jax.ShapeDtypeStruct((n, dout), _F32),
    )(x, w, b.reshape(1, dout))


def _edge_embed(attr, w3, b3, bm):
    nl, din, _ = w3.shape
    e = attr.shape[0]
    nb = e // bm

    def body(a_ref, w_ref, b_ref, o_ref):
        o_ref[...] = (
            jnp.dot(a_ref[...], w_ref[0], preferred_element_type=_F32,
                    precision=lax.Precision.HIGHEST)
            + b_ref[0]
        )[None]

    return pl.pallas_call(
        body,
        grid=(nl, nb),
        in_specs=[
            pl.BlockSpec((bm, din), lambda i, j: (j, 0)),
            pl.BlockSpec((1, din, _D), lambda i, j: (i, 0, 0)),
            pl.BlockSpec((1, 1, _D), lambda i, j: (i, 0, 0)),
        ],
        out_specs=pl.BlockSpec((1, bm, _D), lambda i, j: (i, j, 0)),
        out_shape=jax.ShapeDtypeStruct((nl, e, _D), _F32),
    )(attr, w3, b3)


def _gine_mlp(x, parts, gp, bn, relu_out):
    n = x.shape[0]

    def body(x_ref, p_ref, w1_ref, b1_ref, g1_ref, bb1_ref,
             w2_ref, b2_ref, g2_ref, bb2_ref, o_ref):
        h0 = x_ref[...] + (p_ref[0, :n, :] + p_ref[1, :n, :])
        h = jnp.dot(h0, w1_ref[...], preferred_element_type=_F32) + b1_ref[...]
        mu = jnp.mean(h, axis=0, keepdims=True)
        var = jnp.mean((h - mu) ** 2, axis=0, keepdims=True)
        h = g1_ref[...] * (h - mu) / jnp.sqrt(var + 1e-5) + bb1_ref[...]
        h = jnp.maximum(h, 0.0)
        y = jnp.dot(h, w2_ref[...], preferred_element_type=_F32) + b2_ref[...]
        mu2 = jnp.mean(y, axis=0, keepdims=True)
        var2 = jnp.mean((y - mu2) ** 2, axis=0, keepdims=True)
        y = g2_ref[...] * (y - mu2) / jnp.sqrt(var2 + 1e-5) + bb2_ref[...]
        if relu_out:
            y = jnp.maximum(y, 0.0)
        o_ref[...] = y

    d2 = gp["lin1"]["W"].shape[1]
    return pl.pallas_call(
        body,
        out_shape=jax.ShapeDtypeStruct((n, _D), _F32),
    )(x, parts,
      gp["lin1"]["W"], gp["lin1"]["b"].reshape(1, d2),
      gp["g1"].reshape(1, d2), gp["b1"].reshape(1, d2),
      gp["lin2"]["W"], gp["lin2"]["b"].reshape(1, _D),
      bn["g"].reshape(1, _D), bn["b"].reshape(1, _D))


def _a2f_combine(fx, parts, f_idx2d, wa, ba, n_fg):
    rows, cols = f_idx2d.shape

    def body(fx_ref, p_ref, fi_ref, wa_ref, ba_ref, o_ref):
        sums = p_ref[0, :n_fg, :] + p_ref[1, :n_fg, :]
        iot = lax.broadcasted_iota(jnp.int32, (n_fg, 1), 0)
        cnt = jnp.zeros((n_fg, 1), _F32)
        for j in range(rows):
            blk = fi_ref[j]
            cnt = cnt + jnp.sum((iot == blk[None, :]).astype(_F32),
                                axis=1, keepdims=True)
        mean = sums / jnp.maximum(cnt, 1.0)
        o_ref[...] = (
            fx_ref[...]
            + jnp.dot(mean, wa_ref[...], preferred_element_type=_F32)
            + ba_ref[...]
        )

    return pl.pallas_call(
        body,
        out_shape=jax.ShapeDtypeStruct((n_fg, _D), _F32),
    )(fx, parts, f_idx2d, wa, ba.reshape(1, _D))


def _pool(fx, batch2d, g):
    rows, cols = batch2d.shape

    def body(fx_ref, b_ref, o_ref):
        iot = lax.broadcasted_iota(jnp.int32, (g, 1), 0)
        psum = jnp.zeros((g, _D), _F32)
        pcnt = jnp.zeros((g, 1), _F32)
        for j in range(rows):
            oh = (iot == b_ref[j][None, :]).astype(_F32)
            psum = psum + jnp.dot(oh, fx_ref[pl.ds(j * cols, cols), :],
                                  preferred_element_type=_F32,
                                  precision=lax.Precision.HIGHEST)
            pcnt = pcnt + jnp.sum(oh, axis=1, keepdims=True)
        o_ref[...] = psum / jnp.maximum(pcnt, 1.0)

    n = fx.shape[0]
    return pl.pallas_call(
        body,
        out_shape=jax.ShapeDtypeStruct((g, _D), _F32),
    )(fx, batch2d)


def _sc_segsum(table, src3, dst3, n_acc, ea=None, layer=0):
    nw, nch, k = src3.shape
    epw = nch * k
    rp = n_acc // _NS
    zr = min(k, rp)
    with_ea = ea is not None
    mesh = plsc.VectorSubcoreMesh(core_axis_name="c", subcore_axis_name="s",
                                  num_cores=_NC, num_subcores=_NS)

    def body(*refs):
        if with_ea:
            (src_hbm, dst_hbm, ea_hbm, table_hbm, out_hbm,
             acc_sh, srcall, dstall, rows_v, ea_v, sem) = refs
        else:
            (src_hbm, dst_hbm, table_hbm, out_hbm,
             acc_sh, srcall, dstall, rows_v, sem) = refs
        cid = lax.axis_index("c")
        sid = lax.axis_index("s")
        wid = sid * _NC + cid

        def zrow(r, c_):
            for c in range(_D // 16):
                rows_v[r, pl.ds(c * 16, 16)] = jnp.zeros((16,), _F32)
            return c_
        lax.fori_loop(0, zr, zrow, 0)
        off = 0
        while off < rp:
            take = min(zr, rp - off)
            pltpu.sync_copy(rows_v.at[pl.ds(0, take)],
                            acc_sh.at[pl.ds(sid * rp + off, take)])
            off += take
        plsc.subcore_barrier()

        pltpu.sync_copy(src_hbm.at[wid], srcall)
        pltpu.sync_copy(dst_hbm.at[wid], dstall)

        def chunk(i, c_):
            pltpu.async_copy(table_hbm.at[srcall.at[pl.ds(i * k, k)]],
                             rows_v, sem).wait()
            if with_ea:
                base = wid * epw + i * k
                pltpu.sync_copy(ea_hbm.at[layer, pl.ds(base, k)], ea_v)

                def rowfn(r, c2_):
                    for c in range(_D // 16):
                        s = pl.ds(c * 16, 16)
                        rows_v[r, s] = jnp.maximum(rows_v[r, s] + ea_v[r, s],
                                                   0.0)
                    return c2_
                lax.fori_loop(0, k, rowfn, 0)
            pltpu.sync_copy(rows_v, acc_sh.at[dstall.at[i]], add=True)
            return c_
        lax.fori_loop(0, nch, chunk, 0)

        plsc.subcore_barrier()
        off = 0
        while off < rp:
            take = min(zr, rp - off)
            pltpu.sync_copy(acc_sh.at[pl.ds(sid * rp + off, take)],
                            out_hbm.at[cid, pl.ds(sid * rp + off, take)])
            off += take

    scratch = [
        pltpu.VMEM_SHARED((n_acc, _D), _F32),
        pltpu.VMEM((epw,), jnp.int32),
        pltpu.VMEM((nch, k), jnp.int32),
        pltpu.VMEM((k, _D), _F32),
    ]
    if with_ea:
        scratch.append(pltpu.VMEM((k, _D), _F32))
    scratch.append(pltpu.SemaphoreType.DMA)

    f = pl.kernel(body,
                  out_type=jax.ShapeDtypeStruct((_NC, n_acc, _D), _F32),
                  mesh=mesh, scratch_types=scratch)
    src2 = src3.reshape(nw, epw)
    if with_ea:
        return f(src2, dst3, ea, table)
    return f(src2, dst3, table)


def kernel(atom_x, atom_edge_index, atom_edge_attr, atom_batch, fg_x,
           fg_edge_index, fg_edge_attr, fg_x_batch, atom2fg_index, params):
    p = params
    n_atom = atom_x.shape[0]
    e_atom = atom_edge_index.shape[1]
    n_fg = fg_x.shape[0]
    e_fg = fg_edge_index.shape[1]
    a2f = atom2fg_index.shape[1]
    g = 64

    ax = _linear(atom_x, p["atom_emb"]["W"], p["atom_emb"]["b"])
    fx = _linear(fg_x, p["fg_emb"]["W"], p["fg_emb"]["b"])

    w3 = jnp.stack([l["W"] for l in p["bond_emb"]])
    b3 = jnp.stack([l["b"] for l in p["bond_emb"]])[:, None, :]
    ea3 = _edge_embed(atom_edge_attr, w3, b3, bm=6400)

    wf = jnp.stack([l["W"] for l in p["fg_edge_emb"]])
    bf = jnp.stack([l["b"] for l in p["fg_edge_emb"]])[:, None, :]
    fea2 = _edge_embed(fg_edge_attr, wf, bf, bm=8000)

    k_at = 80
    nch_at = e_atom // _NW // k_at
    src_at = atom_edge_index[0].reshape(_NW, nch_at, k_at)
    dst_at = atom_edge_index[1].reshape(_NW, nch_at, k_at)
    n_acc_at = -(-n_atom // 128) * 128
    for i in range(3):
        parts = _sc_segsum(ax, src_at, dst_at, n_acc_at, ea=ea3, layer=i)
        ax = _gine_mlp(ax, parts, p["atom_gin"][i], p["atom_bn"][i],
                       relu_out=(i != 2))

    k_a2f = 80
    pad = (-a2f) % (_NW * k_a2f)
    a_idx = jnp.concatenate(
        [atom2fg_index[0], jnp.zeros((pad,), jnp.int32)])
    f_idx = jnp.concatenate(
        [atom2fg_index[1], jnp.full((pad,), n_fg, jnp.int32)])
    nch_a2f = (a2f + pad) // _NW // k_a2f
    n_acc = 2048
    parts = _sc_segsum(ax, a_idx.reshape(_NW, nch_a2f, k_a2f),
                       f_idx.reshape(_NW, nch_a2f, k_a2f), n_acc)
    fx = _a2f_combine(fx, parts, atom2fg_index[1].reshape(8, a2f // 8),
                      p["a2f_lin"]["W"], p["a2f_lin"]["b"], n_fg)

    k_fg = 40
    nch_fg = e_fg // _NW // k_fg
    src_fg = fg_edge_index[0].reshape(_NW, nch_fg, k_fg)
    dst_fg = fg_edge_index[1].reshape(_NW, nch_fg, k_fg)
    n_acc_fg = -(-n_fg // 128) * 128
    for i in range(2):
        parts = _sc_segsum(fx, src_fg, dst_fg, n_acc_fg, ea=fea2, layer=i)
        fx = _gine_mlp(fx, parts, p["fg_gin"][i], p["fg_bn"][i],
                       relu_out=(i != 1))

    return _pool(fx, fg_x_batch.reshape(8, n_fg // 8), g)
